# baseline (speedup 1.0000x reference)
import jax
import jax.numpy as jnp
from jax import lax
from jax.experimental import pallas as pl
from jax.experimental.pallas import tpu as pltpu
from jax.experimental.pallas import tpu_sc as plsc

N = 10000
E = 160000
D_IN = 256
D_HID = 512
D_OUT = 256

DH = 128
NP = 10240
NTILES = 16
EDGES_PER_TILE = 10240
EP = NTILES * EDGES_PER_TILE
GBATCH = 64
NBUF = 4
ROWS_PER_TILE = NP // NTILES


def _agg_body(xcat, src2, dst2, dst4, neigh_out, deg_out,
              srcv, dstv, rows0, rows1, rows2, rows3, acc,
              semg0, semg1, semg2, semg3):
    c = lax.axis_index("c")
    s = lax.axis_index("s")
    coff = c * N

    zeros16 = jnp.zeros((16,), jnp.float32)

    def fill_zero(r, _):
        for j in range(DH // 16):
            rows0[r, pl.ds(j * 16, 16)] = zeros16
        return 0

    lax.fori_loop(0, GBATCH, fill_zero, 0)

    rbase = s * ROWS_PER_TILE

    def zbody(j, _):
        pltpu.sync_copy(rows0, acc.at[pl.ds(rbase + j * GBATCH, GBATCH)])
        return 0

    lax.fori_loop(0, ROWS_PER_TILE // GBATCH, zbody, 0)

    plsc.subcore_barrier()

    HE = EDGES_PER_TILE // 2
    HB = HE // GBATCH

    bufs = [rows0, rows1, rows2, rows3]
    sems = [semg0, semg1, semg2, semg3]

    def gstart(b, j):
        pltpu.async_copy(xcat.at[srcv.at[pl.ds(b * GBATCH, GBATCH)]],
                         bufs[j], sems[j])

    def gwait(b, j):
        pltpu.make_async_copy(xcat.at[srcv.at[pl.ds(b * GBATCH, GBATCH)]],
                              bufs[j], sems[j]).wait()

    def scat(b, j):
        pltpu.sync_copy(bufs[j], acc.at[dstv.at[b]], add=True)

    for h in range(2):
        pltpu.sync_copy(src2.at[s, pl.ds(h * HE, HE)], srcv)
        pltpu.sync_copy(dst2.at[s, pl.ds(h * HB, HB)], dstv)

        def adj(i, _):
            srcv[pl.ds(i * 16, 16)] = srcv[pl.ds(i * 16, 16)] + coff
            return 0

        lax.fori_loop(0, HE // 16, adj, 0)

        gstart(0, 0)
        gstart(1, 1)
        gstart(2, 2)

        def body4(i, _):
            for j in range(NBUF):
                b = NBUF * i + j
                gwait(b, j)

                @pl.when(b + 3 < HB)
                def _():
                    gstart(b + 3, (j + 3) % NBUF)

                scat(b, j)
            return 0

        lax.fori_loop(0, HB // NBUF, body4, 0)

    plsc.subcore_barrier()

    pltpu.sync_copy(acc.at[pl.ds(rbase, ROWS_PER_TILE)],
                    neigh_out.at[c, pl.ds(rbase, ROWS_PER_TILE)])

    plsc.subcore_barrier()

    ones16 = jnp.ones((16,), jnp.float32)

    def fill_zo(r, _):
        for j in range(DH // 16):
            rows0[r, pl.ds(j * 16, 16)] = zeros16
            rows1[r, pl.ds(j * 16, 16)] = ones16
        return 0

    lax.fori_loop(0, GBATCH, fill_zo, 0)

    def zbody2(j, _):
        pltpu.sync_copy(rows0, acc.at[pl.ds(rbase + j * GBATCH, GBATCH)])
        return 0

    lax.fori_loop(0, ROWS_PER_TILE // GBATCH, zbody2, 0)

    pltpu.sync_copy(dst4.at[c * NTILES + s], dstv)

    plsc.subcore_barrier()

    def dbody(g, _):
        b = g * 4
        for j in range(4):
            pltpu.async_copy(rows1, acc.at[dstv.at[b + j]], sems[j],
                             add=True)
        for j in range(4):
            pltpu.make_async_copy(rows1, acc.at[dstv.at[b + j]],
                                  sems[j]).wait()
        return 0

    lax.fori_loop(0, (EDGES_PER_TILE // 2) // GBATCH // 4, dbody, 0)

    plsc.subcore_barrier()

    pltpu.sync_copy(acc.at[pl.ds(rbase, ROWS_PER_TILE)],
                    deg_out.at[c, pl.ds(rbase, ROWS_PER_TILE)])


_agg = pl.kernel(
    _agg_body,
    mesh=plsc.VectorSubcoreMesh(core_axis_name="c", subcore_axis_name="s"),
    out_type=[
        jax.ShapeDtypeStruct((2, NP, DH), jnp.float32),
        jax.ShapeDtypeStruct((2, NP, DH), jnp.float32),
    ],
    scratch_types=[
        pltpu.VMEM((EDGES_PER_TILE // 2,), jnp.int32),
        pltpu.VMEM((EDGES_PER_TILE // 2 // GBATCH, GBATCH),
                   jnp.int32),
        pltpu.VMEM((GBATCH, DH), jnp.float32),
        pltpu.VMEM((GBATCH, DH), jnp.float32),
        pltpu.VMEM((GBATCH, DH), jnp.float32),
        pltpu.VMEM((GBATCH, DH), jnp.float32),
        pltpu.VMEM_SHARED((NP, DH), jnp.float32),
        pltpu.SemaphoreType.DMA,
        pltpu.SemaphoreType.DMA,
        pltpu.SemaphoreType.DMA,
        pltpu.SemaphoreType.DMA,
    ],
)


BLK = 1000


def _mlp_body(xb, n2b, db, ws, bs, wn, bn, wo, bo, ob):
    xv = xb[...]
    h_self = jnp.dot(xv, ws[...], preferred_element_type=jnp.float32) + bs[...]
    n2 = n2b[...]
    nb = jnp.concatenate([n2[0], n2[1]], axis=-1)
    d2 = db[...]
    deg = jnp.maximum(d2[0][:, 0:1] + d2[1][:, 0:1], 1.0)
    h_neigh = jnp.dot(nb / deg, wn[...],
                      preferred_element_type=jnp.float32) + bn[...]
    h = jnp.maximum(h_self + h_neigh, 0.0)
    ob[...] = jnp.dot(h, wo[...], preferred_element_type=jnp.float32) + bo[...]


def _mlp(x, neigh2, deg2, ws_t, bs, wn_t, bn, wo_t, bo):
    grid = (N // BLK,)
    return pl.pallas_call(
        _mlp_body,
        grid=grid,
        in_specs=[
            pl.BlockSpec((BLK, D_IN), lambda i: (i, 0)),
            pl.BlockSpec((2, BLK, DH), lambda i: (0, i, 0)),
            pl.BlockSpec((2, BLK, DH), lambda i: (0, i, 0)),
            pl.BlockSpec((D_IN, D_HID), lambda i: (0, 0)),
            pl.BlockSpec((1, D_HID), lambda i: (0, 0)),
            pl.BlockSpec((D_IN, D_HID), lambda i: (0, 0)),
            pl.BlockSpec((1, D_HID), lambda i: (0, 0)),
            pl.BlockSpec((D_HID, D_OUT), lambda i: (0, 0)),
            pl.BlockSpec((1, D_OUT), lambda i: (0, 0)),
        ],
        out_specs=pl.BlockSpec((BLK, D_OUT), lambda i: (i, 0)),
        out_shape=jax.ShapeDtypeStruct((N, D_OUT), jnp.float32),
    )(x, neigh2, deg2, ws_t, bs, wn_t, bn, wo_t, bo)


def kernel(x, edge_index, W_self, b_self, W_neigh, b_neigh, W_out, b_out):
    xcat = jnp.concatenate([x[:, :DH], x[:, DH:]], axis=0)
    src = edge_index[0]
    dst = edge_index[1]
    pad = EP - E
    srcp = jnp.concatenate([src, jnp.zeros((pad,), jnp.int32)])
    dstp = jnp.concatenate([dst, jnp.full((pad,), N, jnp.int32)])
    src2 = srcp.reshape(NTILES, EDGES_PER_TILE)
    dst2 = dstp.reshape(NTILES, EDGES_PER_TILE // GBATCH, GBATCH)
    dst4 = dstp.reshape(2 * NTILES, EDGES_PER_TILE // 2 // GBATCH, GBATCH)
    neigh2, deg2 = _agg(xcat, src2, dst2, dst4)
    return _mlp(x, neigh2, deg2, W_self.T, b_self[None, :],
                W_neigh.T, b_neigh[None, :], W_out.T, b_out[None, :])


# speedup vs baseline: 1.0114x; 1.0114x over previous
import jax
import jax.numpy as jnp
from jax import lax
from jax.experimental import pallas as pl
from jax.experimental.pallas import tpu as pltpu
from jax.experimental.pallas import tpu_sc as plsc

N = 10000
E = 160000
D_IN = 256
D_HID = 512
D_OUT = 256

DH = 128
NP = 10240
NTILES = 16
EDGES_PER_TILE = 10240
EP = NTILES * EDGES_PER_TILE
BATCH = 128
GBATCH = 64
NBUF = 4
ROWS_PER_TILE = NP // NTILES

DEG_EDGES_PER_TILE = EP // 32
DEG_NBATCH = DEG_EDGES_PER_TILE // BATCH


def _agg_body(xcat, src2, dst2, neigh_out,
              srcv, dstv, rows0, rows1, rows2, rows3, acc,
              semg0, semg1, semg2, semg3):
    c = lax.axis_index("c")
    s = lax.axis_index("s")
    coff = c * N

    zeros16 = jnp.zeros((16,), jnp.float32)

    def fill_zero(r, _):
        for j in range(DH // 16):
            rows0[r, pl.ds(j * 16, 16)] = zeros16
        return 0

    lax.fori_loop(0, GBATCH, fill_zero, 0)

    rbase = s * ROWS_PER_TILE

    def zbody(j, _):
        pltpu.sync_copy(rows0, acc.at[pl.ds(rbase + j * GBATCH, GBATCH)])
        return 0

    lax.fori_loop(0, ROWS_PER_TILE // GBATCH, zbody, 0)

    plsc.subcore_barrier()

    HE = EDGES_PER_TILE // 2
    HB = HE // GBATCH

    bufs = [rows0, rows1, rows2, rows3]
    sems = [semg0, semg1, semg2, semg3]

    def gstart(b, j):
        pltpu.async_copy(xcat.at[srcv.at[pl.ds(b * GBATCH, GBATCH)]],
                         bufs[j], sems[j])

    def gwait(b, j):
        pltpu.make_async_copy(xcat.at[srcv.at[pl.ds(b * GBATCH, GBATCH)]],
                              bufs[j], sems[j]).wait()

    def scat(b, j):
        pltpu.sync_copy(bufs[j], acc.at[dstv.at[b]], add=True)

    for h in range(2):
        pltpu.sync_copy(src2.at[s, pl.ds(h * HE, HE)], srcv)
        pltpu.sync_copy(dst2.at[s, pl.ds(h * HB, HB)], dstv)

        def adj(i, _):
            srcv[pl.ds(i * 16, 16)] = srcv[pl.ds(i * 16, 16)] + coff
            return 0

        lax.fori_loop(0, HE // 16, adj, 0)

        gstart(0, 0)
        gstart(1, 1)
        gstart(2, 2)

        def body4(i, _):
            for j in range(NBUF):
                b = NBUF * i + j
                gwait(b, j)

                @pl.when(b + 3 < HB)
                def _():
                    gstart(b + 3, (j + 3) % NBUF)

                scat(b, j)
            return 0

        lax.fori_loop(0, HB // NBUF, body4, 0)

    plsc.subcore_barrier()

    pltpu.sync_copy(acc.at[pl.ds(rbase, ROWS_PER_TILE)],
                    neigh_out.at[c, pl.ds(rbase, ROWS_PER_TILE)])


_agg = pl.kernel(
    _agg_body,
    mesh=plsc.VectorSubcoreMesh(core_axis_name="c", subcore_axis_name="s"),
    out_type=[
        jax.ShapeDtypeStruct((2, NP, DH), jnp.float32),
    ],
    scratch_types=[
        pltpu.VMEM((EDGES_PER_TILE // 2,), jnp.int32),
        pltpu.VMEM((EDGES_PER_TILE // 2 // GBATCH, GBATCH),
                   jnp.int32),
        pltpu.VMEM((GBATCH, DH), jnp.float32),
        pltpu.VMEM((GBATCH, DH), jnp.float32),
        pltpu.VMEM((GBATCH, DH), jnp.float32),
        pltpu.VMEM((GBATCH, DH), jnp.float32),
        pltpu.VMEM_SHARED((NP, DH), jnp.float32),
        pltpu.SemaphoreType.DMA,
        pltpu.SemaphoreType.DMA,
        pltpu.SemaphoreType.DMA,
        pltpu.SemaphoreType.DMA,
    ],
)


def _deg_body(dst3, deg_out, dstv, onesv, zdv, dacc, semd):
    c = lax.axis_index("c")
    s = lax.axis_index("s")

    zeros16 = jnp.zeros((16,), jnp.float32)
    ones16 = jnp.ones((16,), jnp.float32)

    def fill(r, _):
        for j in range(DH // 16):
            zdv[r, pl.ds(j * 16, 16)] = zeros16
        return 0

    lax.fori_loop(0, 16, fill, 0)

    def fill_ones(r, _):
        for j in range(DH // 16):
            onesv[r, pl.ds(j * 16, 16)] = ones16
        return 0

    lax.fori_loop(0, BATCH, fill_ones, 0)

    pltpu.sync_copy(dst3.at[c * NTILES + s], dstv)

    rbase = s * ROWS_PER_TILE

    def zbody(j, _):
        pltpu.sync_copy(zdv, dacc.at[pl.ds(rbase + j * 16, 16)])
        return 0

    lax.fori_loop(0, ROWS_PER_TILE // 16, zbody, 0)

    plsc.subcore_barrier()

    def body(g, _):
        b = g * 4
        for j in range(4):
            pltpu.async_copy(onesv, dacc.at[dstv.at[b + j]], semd, add=True)
        for j in range(4):
            pltpu.make_async_copy(onesv, dacc.at[dstv.at[b + j]],
                                  semd).wait()
        return 0

    lax.fori_loop(0, DEG_NBATCH // 4, body, 0)

    plsc.subcore_barrier()

    pltpu.sync_copy(dacc.at[pl.ds(rbase, ROWS_PER_TILE)],
                    deg_out.at[c, pl.ds(rbase, ROWS_PER_TILE)])


_deg = pl.kernel(
    _deg_body,
    mesh=plsc.VectorSubcoreMesh(core_axis_name="c", subcore_axis_name="s"),
    out_type=[
        jax.ShapeDtypeStruct((2, NP, DH), jnp.float32),
    ],
    scratch_types=[
        pltpu.VMEM((DEG_NBATCH, BATCH), jnp.int32),
        pltpu.VMEM((BATCH, DH), jnp.float32),
        pltpu.VMEM((16, DH), jnp.float32),
        pltpu.VMEM_SHARED((NP, DH), jnp.float32),
        pltpu.SemaphoreType.DMA,
    ],
)


BLK = 1000


def _mlp_body(xb, n2b, db, ws, bs, wn, bn, wo, bo, ob):
    xv = xb[...]
    h_self = jnp.dot(xv, ws[...], preferred_element_type=jnp.float32) + bs[...]
    n2 = n2b[...]
    nb = jnp.concatenate([n2[0], n2[1]], axis=-1)
    d2 = db[...]
    deg = jnp.maximum(d2[0][:, 0:1] + d2[1][:, 0:1], 1.0)
    h_neigh = jnp.dot(nb / deg, wn[...],
                      preferred_element_type=jnp.float32) + bn[...]
    h = jnp.maximum(h_self + h_neigh, 0.0)
    ob[...] = jnp.dot(h, wo[...], preferred_element_type=jnp.float32) + bo[...]


def _mlp(x, neigh2, deg2, ws_t, bs, wn_t, bn, wo_t, bo):
    grid = (N // BLK,)
    return pl.pallas_call(
        _mlp_body,
        grid=grid,
        in_specs=[
            pl.BlockSpec((BLK, D_IN), lambda i: (i, 0)),
            pl.BlockSpec((2, BLK, DH), lambda i: (0, i, 0)),
            pl.BlockSpec((2, BLK, DH), lambda i: (0, i, 0)),
            pl.BlockSpec((D_IN, D_HID), lambda i: (0, 0)),
            pl.BlockSpec((1, D_HID), lambda i: (0, 0)),
            pl.BlockSpec((D_IN, D_HID), lambda i: (0, 0)),
            pl.BlockSpec((1, D_HID), lambda i: (0, 0)),
            pl.BlockSpec((D_HID, D_OUT), lambda i: (0, 0)),
            pl.BlockSpec((1, D_OUT), lambda i: (0, 0)),
        ],
        out_specs=pl.BlockSpec((BLK, D_OUT), lambda i: (i, 0)),
        out_shape=jax.ShapeDtypeStruct((N, D_OUT), jnp.float32),
    )(x, neigh2, deg2, ws_t, bs, wn_t, bn, wo_t, bo)


def kernel(x, edge_index, W_self, b_self, W_neigh, b_neigh, W_out, b_out):
    xcat = jnp.concatenate([x[:, :DH], x[:, DH:]], axis=0)
    src = edge_index[0]
    dst = edge_index[1]
    pad = EP - E
    srcp = jnp.concatenate([src, jnp.zeros((pad,), jnp.int32)])
    dstp = jnp.concatenate([dst, jnp.full((pad,), N, jnp.int32)])
    src2 = srcp.reshape(NTILES, EDGES_PER_TILE)
    dst2 = dstp.reshape(NTILES, EDGES_PER_TILE // GBATCH, GBATCH)
    dst3 = dstp.reshape(2 * NTILES, DEG_NBATCH, BATCH)
    (neigh2,) = _agg(xcat, src2, dst2)
    (deg2,) = _deg(dst3)
    return _mlp(x, neigh2, deg2, W_self.T, b_self[None, :],
                W_neigh.T, b_neigh[None, :], W_out.T, b_out[None, :])


# speedup vs baseline: 1.0530x; 1.0411x over previous
import jax
import jax.numpy as jnp
from jax import lax
from jax.experimental import pallas as pl
from jax.experimental.pallas import tpu as pltpu
from jax.experimental.pallas import tpu_sc as plsc

N = 10000
E = 160000
D_IN = 256
D_HID = 512
D_OUT = 256

DH = 128
NP = 10240
NTILES = 16
EDGES_PER_TILE = 10240
EP = NTILES * EDGES_PER_TILE
BATCH = 128
GBATCH = 128
NBUF = 2
ROWS_PER_TILE = NP // NTILES

DEG_EDGES_PER_TILE = EP // 32
DEG_NBATCH = DEG_EDGES_PER_TILE // BATCH


def _agg_body(xcat, src2, dst2, neigh_out,
              srcv, dstv, rows0, rows1, acc, semg0, semg1):
    c = lax.axis_index("c")
    s = lax.axis_index("s")
    coff = c * N

    zeros16 = jnp.zeros((16,), jnp.float32)

    def fill_zero(r, _):
        for j in range(DH // 16):
            rows0[r, pl.ds(j * 16, 16)] = zeros16
        return 0

    lax.fori_loop(0, GBATCH, fill_zero, 0)

    rbase = s * ROWS_PER_TILE

    def zbody(j, _):
        pltpu.sync_copy(rows0, acc.at[pl.ds(rbase + j * GBATCH, GBATCH)])
        return 0

    lax.fori_loop(0, ROWS_PER_TILE // GBATCH, zbody, 0)

    plsc.subcore_barrier()

    HE = EDGES_PER_TILE // 2
    HB = HE // GBATCH

    bufs = [rows0, rows1]
    sems = [semg0, semg1]

    def gstart(b, j):
        pltpu.async_copy(xcat.at[srcv.at[pl.ds(b * GBATCH, GBATCH)]],
                         bufs[j], sems[j])

    def gwait(b, j):
        pltpu.make_async_copy(xcat.at[srcv.at[pl.ds(b * GBATCH, GBATCH)]],
                              bufs[j], sems[j]).wait()

    def scat(b, j):
        pltpu.sync_copy(bufs[j], acc.at[dstv.at[b]], add=True)

    for h in range(2):
        pltpu.sync_copy(src2.at[s, pl.ds(h * HE, HE)], srcv)
        pltpu.sync_copy(dst2.at[s, pl.ds(h * HB, HB)], dstv)

        def adj(i, _):
            srcv[pl.ds(i * 16, 16)] = srcv[pl.ds(i * 16, 16)] + coff
            return 0

        lax.fori_loop(0, HE // 16, adj, 0)

        gstart(0, 0)
        gstart(1, 1)

        def body2(i, _):
            for j in range(NBUF):
                b = NBUF * i + j
                gwait(b, j)

                @pl.when(b + 2 < HB)
                def _():
                    gstart(b + 2, j)

                scat(b, j)
            return 0

        lax.fori_loop(0, HB // NBUF, body2, 0)

    plsc.subcore_barrier()

    pltpu.sync_copy(acc.at[pl.ds(rbase, ROWS_PER_TILE)],
                    neigh_out.at[c, pl.ds(rbase, ROWS_PER_TILE)])


_agg = pl.kernel(
    _agg_body,
    mesh=plsc.VectorSubcoreMesh(core_axis_name="c", subcore_axis_name="s"),
    out_type=[
        jax.ShapeDtypeStruct((2, NP, DH), jnp.float32),
    ],
    scratch_types=[
        pltpu.VMEM((EDGES_PER_TILE // 2,), jnp.int32),
        pltpu.VMEM((EDGES_PER_TILE // 2 // GBATCH, GBATCH),
                   jnp.int32),
        pltpu.VMEM((GBATCH, DH), jnp.float32),
        pltpu.VMEM((GBATCH, DH), jnp.float32),
        pltpu.VMEM_SHARED((NP, DH), jnp.float32),
        pltpu.SemaphoreType.DMA,
        pltpu.SemaphoreType.DMA,
    ],
)


def _deg_body(dst3, deg_out, dstv, onesv, zdv, dacc, semd):
    c = lax.axis_index("c")
    s = lax.axis_index("s")

    zeros16 = jnp.zeros((16,), jnp.float32)
    ones16 = jnp.ones((16,), jnp.float32)

    def fill(r, _):
        for j in range(DH // 16):
            zdv[r, pl.ds(j * 16, 16)] = zeros16
        return 0

    lax.fori_loop(0, 16, fill, 0)

    def fill_ones(r, _):
        for j in range(DH // 16):
            onesv[r, pl.ds(j * 16, 16)] = ones16
        return 0

    lax.fori_loop(0, BATCH, fill_ones, 0)

    pltpu.sync_copy(dst3.at[c * NTILES + s], dstv)

    rbase = s * ROWS_PER_TILE

    def zbody(j, _):
        pltpu.sync_copy(zdv, dacc.at[pl.ds(rbase + j * 16, 16)])
        return 0

    lax.fori_loop(0, ROWS_PER_TILE // 16, zbody, 0)

    plsc.subcore_barrier()

    def body(g, _):
        b = g * 4
        for j in range(4):
            pltpu.async_copy(onesv, dacc.at[dstv.at[b + j]], semd, add=True)
        for j in range(4):
            pltpu.make_async_copy(onesv, dacc.at[dstv.at[b + j]],
                                  semd).wait()
        return 0

    lax.fori_loop(0, DEG_NBATCH // 4, body, 0)

    plsc.subcore_barrier()

    pltpu.sync_copy(dacc.at[pl.ds(rbase, ROWS_PER_TILE)],
                    deg_out.at[c, pl.ds(rbase, ROWS_PER_TILE)])


_deg = pl.kernel(
    _deg_body,
    mesh=plsc.VectorSubcoreMesh(core_axis_name="c", subcore_axis_name="s"),
    out_type=[
        jax.ShapeDtypeStruct((2, NP, DH), jnp.float32),
    ],
    scratch_types=[
        pltpu.VMEM((DEG_NBATCH, BATCH), jnp.int32),
        pltpu.VMEM((BATCH, DH), jnp.float32),
        pltpu.VMEM((16, DH), jnp.float32),
        pltpu.VMEM_SHARED((NP, DH), jnp.float32),
        pltpu.SemaphoreType.DMA,
    ],
)


BLK = 1000


def _mlp_body(xb, n2b, db, ws, bs, wn, bn, wo, bo, ob):
    xv = xb[...]
    h_self = jnp.dot(xv, ws[...], preferred_element_type=jnp.float32) + bs[...]
    n2 = n2b[...]
    nb = jnp.concatenate([n2[0], n2[1]], axis=-1)
    d2 = db[...]
    deg = jnp.maximum(d2[0][:, 0:1] + d2[1][:, 0:1], 1.0)
    h_neigh = jnp.dot(nb / deg, wn[...],
                      preferred_element_type=jnp.float32) + bn[...]
    h = jnp.maximum(h_self + h_neigh, 0.0)
    ob[...] = jnp.dot(h, wo[...], preferred_element_type=jnp.float32) + bo[...]


def _mlp(x, neigh2, deg2, ws_t, bs, wn_t, bn, wo_t, bo):
    grid = (N // BLK,)
    return pl.pallas_call(
        _mlp_body,
        grid=grid,
        in_specs=[
            pl.BlockSpec((BLK, D_IN), lambda i: (i, 0)),
            pl.BlockSpec((2, BLK, DH), lambda i: (0, i, 0)),
            pl.BlockSpec((2, BLK, DH), lambda i: (0, i, 0)),
            pl.BlockSpec((D_IN, D_HID), lambda i: (0, 0)),
            pl.BlockSpec((1, D_HID), lambda i: (0, 0)),
            pl.BlockSpec((D_IN, D_HID), lambda i: (0, 0)),
            pl.BlockSpec((1, D_HID), lambda i: (0, 0)),
            pl.BlockSpec((D_HID, D_OUT), lambda i: (0, 0)),
            pl.BlockSpec((1, D_OUT), lambda i: (0, 0)),
        ],
        out_specs=pl.BlockSpec((BLK, D_OUT), lambda i: (i, 0)),
        out_shape=jax.ShapeDtypeStruct((N, D_OUT), jnp.float32),
    )(x, neigh2, deg2, ws_t, bs, wn_t, bn, wo_t, bo)


def kernel(x, edge_index, W_self, b_self, W_neigh, b_neigh, W_out, b_out):
    xcat = jnp.concatenate([x[:, :DH], x[:, DH:]], axis=0)
    src = edge_index[0]
    dst = edge_index[1]
    pad = EP - E
    srcp = jnp.concatenate([src, jnp.zeros((pad,), jnp.int32)])
    dstp = jnp.concatenate([dst, jnp.full((pad,), N, jnp.int32)])
    src2 = srcp.reshape(NTILES, EDGES_PER_TILE)
    dst2 = dstp.reshape(NTILES, EDGES_PER_TILE // GBATCH, GBATCH)
    dst3 = dstp.reshape(2 * NTILES, DEG_NBATCH, BATCH)
    (neigh2,) = _agg(xcat, src2, dst2)
    (deg2,) = _deg(dst3)
    return _mlp(x, neigh2, deg2, W_self.T, b_self[None, :],
                W_neigh.T, b_neigh[None, :], W_out.T, b_out[None, :])
